# baseline (speedup 1.0000x reference)
import functools

import jax
import jax.numpy as jnp
from jax import lax
from jax.experimental import pallas as pl
from jax.experimental.pallas import tpu as pltpu
from jax.experimental.pallas import tpu_sc as plsc

N = 320000
D = 128
S = 10000
L = 16
NW = 32
CHUNK = N // NW
BN = 32000

_mesh = plsc.VectorSubcoreMesh(core_axis_name="c", subcore_axis_name="s")
_sc_params = pltpu.CompilerParams(needs_layout_passes=False)


def _gate_body(fea_ref, w_ref, gate_ref, max_ref):
    g = jnp.dot(fea_ref[...], w_ref[...], preferred_element_type=jnp.float32)
    g128 = g.reshape(1, BN // 128, 128)
    gate_ref[...] = g128
    bm = jnp.max(g128)

    @pl.when(pl.program_id(0) == 0)
    def _():
        max_ref[...] = jnp.full((8, 128), bm, jnp.float32)

    @pl.when(pl.program_id(0) > 0)
    def _():
        max_ref[...] = jnp.maximum(max_ref[...], bm)


def _tc_gate(fea, W):
    return pl.pallas_call(
        _gate_body,
        grid=(N // BN,),
        in_specs=[
            pl.BlockSpec((BN, D), lambda i: (i, 0)),
            pl.BlockSpec((D, 1), lambda i: (0, 0)),
        ],
        out_specs=[
            pl.BlockSpec((1, BN // 128, 128), lambda i: (i, 0, 0)),
            pl.BlockSpec((8, 128), lambda i: (0, 0)),
        ],
        out_shape=[
            jax.ShapeDtypeStruct((N // BN, BN // 128, 128), jnp.float32),
            jax.ShapeDtypeStruct((8, 128), jnp.float32),
        ],
    )(fea, W)


@functools.partial(
    pl.kernel,
    mesh=_mesh,
    out_type=jax.ShapeDtypeStruct((NW, S), jnp.float32),
    scratch_types=[
        pltpu.VMEM((CHUNK,), jnp.float32),
        pltpu.VMEM((CHUNK,), jnp.int32),
        pltpu.VMEM((S,), jnp.float32),
        pltpu.VMEM((L,), jnp.float32),
    ],
    compiler_params=_sc_params,
)
def _sc_exp_scatter(gate_hbm, idx_hbm, gmax_hbm, tab_hbm, g_v, i_v, t_v, m_v):
    wid = lax.axis_index("s") * 2 + lax.axis_index("c")
    base = wid * CHUNK
    pltpu.sync_copy(gate_hbm.at[pl.ds(base, CHUNK)], g_v)
    pltpu.sync_copy(idx_hbm.at[pl.ds(base, CHUNK)], i_v)
    pltpu.sync_copy(gmax_hbm, m_v)
    m = m_v[...]

    @plsc.parallel_loop(0, S, step=L, unroll=16)
    def _zero(j):
        t_v[pl.ds(j, L)] = jnp.zeros((L,), jnp.float32)

    iota = lax.iota(jnp.int32, L)
    shift_r = jnp.maximum(iota - 1, 0)
    shift_l = jnp.minimum(iota + 1, L - 1)

    @plsc.parallel_loop(0, CHUNK, step=L, unroll=16)
    def _scatter(j):
        g = g_v[pl.ds(j, L)]
        ix = i_v[pl.ds(j, L)]
        e = jnp.exp(g - m)
        c = plsc.cumsum(e)
        cex = c - e
        start = (ix != ix[shift_r]) | (iota == 0)
        sp = plsc.cummax(jnp.where(start, iota, 0))
        run = c - cex[sp]
        end = (ix != ix[shift_l]) | (iota == L - 1)
        plsc.addupdate_scatter(t_v, [ix], run, mask=end)

    pltpu.sync_copy(t_v, tab_hbm.at[wid])


def _reduce_body(tab_ref, tot_ref):
    tot_ref[...] = 1.0 / (jnp.sum(tab_ref[...], axis=0) + 1e-13)


def _tc_reduce(tabs):
    return pl.pallas_call(
        _reduce_body,
        out_shape=jax.ShapeDtypeStruct((S,), jnp.float32),
    )(tabs)


@functools.partial(
    pl.kernel,
    mesh=_mesh,
    out_type=jax.ShapeDtypeStruct((N,), jnp.float32),
    scratch_types=[
        pltpu.VMEM((CHUNK,), jnp.float32),
        pltpu.VMEM((CHUNK,), jnp.int32),
        pltpu.VMEM((S,), jnp.float32),
        pltpu.VMEM((CHUNK,), jnp.float32),
        pltpu.VMEM((L,), jnp.float32),
    ],
    compiler_params=_sc_params,
)
def _sc_apply(gate_hbm, idx_hbm, tot_hbm, gmax_hbm, out_hbm,
              g_v, i_v, t_v, o_v, m_v):
    wid = lax.axis_index("s") * 2 + lax.axis_index("c")
    base = wid * CHUNK
    pltpu.sync_copy(gate_hbm.at[pl.ds(base, CHUNK)], g_v)
    pltpu.sync_copy(idx_hbm.at[pl.ds(base, CHUNK)], i_v)
    pltpu.sync_copy(tot_hbm, t_v)
    pltpu.sync_copy(gmax_hbm, m_v)
    m = m_v[...]

    @plsc.parallel_loop(0, CHUNK, step=L, unroll=16)
    def _apply(j):
        g = g_v[pl.ds(j, L)]
        ix = i_v[pl.ds(j, L)]
        e = jnp.exp(g - m)
        rinv = plsc.load_gather(t_v, [ix])
        o_v[pl.ds(j, L)] = e * rinv

    pltpu.sync_copy(o_v, out_hbm.at[pl.ds(base, CHUNK)])


def kernel(fea, index, W, b):
    del b
    gate, gmax = _tc_gate(fea, W)
    gate1 = gate.reshape(N)
    gmax16 = gmax[0, :L]
    tabs = _sc_exp_scatter(gate1, index, gmax16)
    tot = _tc_reduce(tabs)
    return _sc_apply(gate1, index, tot, gmax16).reshape(N, 1)


# speedup vs baseline: 1.0256x; 1.0256x over previous
import functools

import jax
import jax.numpy as jnp
from jax import lax
from jax.experimental import pallas as pl
from jax.experimental.pallas import tpu as pltpu
from jax.experimental.pallas import tpu_sc as plsc

N = 320000
D = 128
S = 10000
L = 16
NW = 32
CHUNK = N // NW
BN = 32000

_mesh = plsc.VectorSubcoreMesh(core_axis_name="c", subcore_axis_name="s")
_sc_params = pltpu.CompilerParams(needs_layout_passes=False)


def _gate_body(fea_ref, w_ref, gate_ref, max_ref):
    g = jnp.dot(fea_ref[...], w_ref[...], preferred_element_type=jnp.float32)
    g128 = g.reshape(1, BN // 128, 128)
    gate_ref[...] = g128
    bm = jnp.max(g128)

    @pl.when(pl.program_id(0) == 0)
    def _():
        max_ref[...] = jnp.full((8, 128), bm, jnp.float32)

    @pl.when(pl.program_id(0) > 0)
    def _():
        max_ref[...] = jnp.maximum(max_ref[...], bm)


def _tc_gate(fea, W):
    return pl.pallas_call(
        _gate_body,
        grid=(N // BN,),
        in_specs=[
            pl.BlockSpec((BN, D), lambda i: (i, 0)),
            pl.BlockSpec((D, 1), lambda i: (0, 0)),
        ],
        out_specs=[
            pl.BlockSpec((1, BN // 128, 128), lambda i: (i, 0, 0)),
            pl.BlockSpec((8, 128), lambda i: (0, 0)),
        ],
        out_shape=[
            jax.ShapeDtypeStruct((N // BN, BN // 128, 128), jnp.float32),
            jax.ShapeDtypeStruct((8, 128), jnp.float32),
        ],
    )(fea, W)


@functools.partial(
    pl.kernel,
    mesh=_mesh,
    out_type=jax.ShapeDtypeStruct((NW, S), jnp.float32),
    scratch_types=[
        pltpu.VMEM((CHUNK,), jnp.float32),
        pltpu.VMEM((CHUNK,), jnp.int32),
        pltpu.VMEM((S,), jnp.float32),
        pltpu.VMEM((L,), jnp.float32),
    ],
    compiler_params=_sc_params,
)
def _sc_exp_scatter(gate_hbm, idx_hbm, gmax_hbm, tab_hbm, g_v, i_v, t_v, m_v):
    wid = lax.axis_index("s") * 2 + lax.axis_index("c")
    base = wid * CHUNK
    pltpu.sync_copy(gate_hbm.at[pl.ds(base, CHUNK)], g_v)
    pltpu.sync_copy(idx_hbm.at[pl.ds(base, CHUNK)], i_v)
    pltpu.sync_copy(gmax_hbm, m_v)
    m = m_v[...]

    @plsc.parallel_loop(0, S, step=L, unroll=8)
    def _zero(j):
        t_v[pl.ds(j, L)] = jnp.zeros((L,), jnp.float32)

    iota = lax.iota(jnp.int32, L)
    shift_r = jnp.maximum(iota - 1, 0)
    shift_l = jnp.minimum(iota + 1, L - 1)

    @plsc.parallel_loop(0, CHUNK, step=L, unroll=8)
    def _scatter(j):
        g = g_v[pl.ds(j, L)]
        ix = i_v[pl.ds(j, L)]
        e = jnp.exp(g - m)
        c = plsc.cumsum(e)
        cex = c - e
        start = (ix != ix[shift_r]) | (iota == 0)
        sp = plsc.cummax(jnp.where(start, iota, 0))
        run = c - cex[sp]
        end = (ix != ix[shift_l]) | (iota == L - 1)
        plsc.addupdate_scatter(t_v, [ix], run, mask=end)

    pltpu.sync_copy(t_v, tab_hbm.at[wid])


def _reduce_body(tab_ref, tot_ref):
    tot_ref[...] = 1.0 / (jnp.sum(tab_ref[...], axis=0) + 1e-13)


def _tc_reduce(tabs):
    return pl.pallas_call(
        _reduce_body,
        out_shape=jax.ShapeDtypeStruct((S,), jnp.float32),
    )(tabs)


@functools.partial(
    pl.kernel,
    mesh=_mesh,
    out_type=jax.ShapeDtypeStruct((N,), jnp.float32),
    scratch_types=[
        pltpu.VMEM((CHUNK,), jnp.float32),
        pltpu.VMEM((CHUNK,), jnp.int32),
        pltpu.VMEM((S,), jnp.float32),
        pltpu.VMEM((CHUNK,), jnp.float32),
        pltpu.VMEM((L,), jnp.float32),
    ],
    compiler_params=_sc_params,
)
def _sc_apply(gate_hbm, idx_hbm, tot_hbm, gmax_hbm, out_hbm,
              g_v, i_v, t_v, o_v, m_v):
    wid = lax.axis_index("s") * 2 + lax.axis_index("c")
    base = wid * CHUNK
    pltpu.sync_copy(gate_hbm.at[pl.ds(base, CHUNK)], g_v)
    pltpu.sync_copy(idx_hbm.at[pl.ds(base, CHUNK)], i_v)
    pltpu.sync_copy(tot_hbm, t_v)
    pltpu.sync_copy(gmax_hbm, m_v)
    m = m_v[...]

    @plsc.parallel_loop(0, CHUNK, step=L, unroll=8)
    def _apply(j):
        g = g_v[pl.ds(j, L)]
        ix = i_v[pl.ds(j, L)]
        e = jnp.exp(g - m)
        rinv = plsc.load_gather(t_v, [ix])
        o_v[pl.ds(j, L)] = e * rinv

    pltpu.sync_copy(o_v, out_hbm.at[pl.ds(base, CHUNK)])


def kernel(fea, index, W, b):
    del b
    gate, gmax = _tc_gate(fea, W)
    gate1 = gate.reshape(N)
    gmax16 = gmax[0, :L]
    tabs = _sc_exp_scatter(gate1, index, gmax16)
    tot = _tc_reduce(tabs)
    return _sc_apply(gate1, index, tot, gmax16).reshape(N, 1)
